# baseline (speedup 1.0000x reference)
import functools

import jax
import jax.numpy as jnp
from jax import lax
from jax.experimental import pallas as pl
from jax.experimental.pallas import tpu as pltpu
from jax.experimental.pallas import tpu_sc as plsc

N = 10000
NPAD = 10240
D = 256
DH = 128
E = 160000
EPAD = 163840
CH = 32
NCHUNK = 320
NLOAD = 40
RING = 8
RINGF = 7
ROWS_PER_TILE = NPAD // 16
B = 50000
BPAD = 53248
PPTF = 3328
CHF = 64
NCF = 52
DP = 128

f32 = jnp.float32
bf16 = jnp.bfloat16
i32 = jnp.int32

_MESH = plsc.VectorSubcoreMesh(core_axis_name="c", subcore_axis_name="s")


def _make_sc_agg(with_cnt):
    out_type = [jax.ShapeDtypeStruct((2, NPAD, DH), f32)]
    if with_cnt:
        out_type += [jax.ShapeDtypeStruct((NPAD,), f32),
                     jax.ShapeDtypeStruct((NPAD,), f32)]
    scratch = [
        pltpu.VMEM_SHARED((NPAD, DH), f32),
        pltpu.VMEM((NLOAD, CH), i32),
        pltpu.VMEM((NLOAD, CH), i32),
        pltpu.VMEM((RING, CH, DH), f32),
        pltpu.SemaphoreType.DMA,
        pltpu.SemaphoreType.DMA,
    ]
    if with_cnt:
        scratch += [
            pltpu.VMEM_SHARED((NPAD,), f32),
            pltpu.VMEM((CH,), f32),
            pltpu.SemaphoreType.DMA,
        ]

    @functools.partial(pl.kernel, out_type=out_type, mesh=_MESH,
                       scratch_types=scratch)
    def k(table, gidx, sidx, *rest):
        if with_cnt:
            agg_out, cnt_s_out, cnt_g_out = rest[0], rest[1], rest[2]
            acc, gv, sv, rb, sem_g, sem_s, cntacc, ones_v, sem_c = rest[3:]
        else:
            agg_out = rest[0]
            acc, gv, sv, rb, sem_g, sem_s = rest[1:]
            cntacc = ones_v = sem_c = None
        c = lax.axis_index("c")
        s = lax.axis_index("s")
        base = s * ROWS_PER_TILE

        def zrow(j, carry):
            rb[0, j // 8, pl.ds((j % 8) * 16, 16)] = jnp.zeros((16,), f32)
            return carry
        lax.fori_loop(0, (CH * DH) // 16, zrow, 0)
        nz = ROWS_PER_TILE // CH
        for q in range(nz):
            pltpu.make_async_copy(
                rb.at[0], acc.at[pl.ds(base + q * CH, CH)], sem_g).start()
        for q in range(nz):
            pltpu.make_async_copy(
                rb.at[0], acc.at[pl.ds(base + q * CH, CH)], sem_g).wait()
        if with_cnt:
            for q in range(ROWS_PER_TILE // DH):
                pltpu.sync_copy(rb.at[0].at[0],
                                cntacc.at[pl.ds(base + q * DH, DH)])
            for q in range(CH // 16):
                ones_v[pl.ds(q * 16, 16)] = jnp.ones((16,), f32)

        def gdesc(kk, u):
            return pltpu.make_async_copy(table.at[c].at[gv.at[kk]],
                                         rb.at[u], sem_g)

        def sdesc(kk, u):
            return pltpu.make_async_copy(rb.at[u], acc.at[sv.at[kk]], sem_s)

        def cdesc(kk, use_sv):
            idxv = sv if use_sv else gv
            return pltpu.make_async_copy(ones_v, cntacc.at[idxv.at[kk]],
                                         sem_c)

        AHEAD = RING - 3

        def step(j, carry):
            for u in range(RING):
                kk = j * RING + u
                gdesc(kk, u).wait()
                sdesc(kk, u).start(add=True)
                if with_cnt:
                    @pl.when(c == 0)
                    def _():
                        cdesc(kk, True).start(add=True)

                    @pl.when(c == 1)
                    def _():
                        cdesc(kk, False).start(add=True)

                    @pl.when(kk >= 1)
                    def _():
                        @pl.when(c == 0)
                        def _():
                            cdesc(kk - 1, True).wait()

                        @pl.when(c == 1)
                        def _():
                            cdesc(kk - 1, False).wait()

                @pl.when(kk >= 3)
                def _():
                    sdesc(kk - 3, (u - 3) % RING).wait()

                @pl.when(kk < NLOAD - AHEAD)
                def _():
                    gdesc(kk + AHEAD, (u + AHEAD) % RING).start()
            return carry

        for h in range(NCHUNK // NLOAD):
            pltpu.sync_copy(gidx.at[s].at[pl.ds(h * NLOAD, NLOAD)], gv)
            pltpu.sync_copy(sidx.at[s].at[pl.ds(h * NLOAD, NLOAD)], sv)
            if h == 0:
                plsc.subcore_barrier()
            for p in range(AHEAD):
                gdesc(p, p).start()
            lax.fori_loop(0, NLOAD // RING, step, 0)
            sdesc(NLOAD - 3, (NLOAD - 3) % RING).wait()
            sdesc(NLOAD - 2, (NLOAD - 2) % RING).wait()
            sdesc(NLOAD - 1, (NLOAD - 1) % RING).wait()
            if with_cnt:
                @pl.when(c == 0)
                def _():
                    cdesc(NLOAD - 1, True).wait()

                @pl.when(c == 1)
                def _():
                    cdesc(NLOAD - 1, False).wait()
        plsc.subcore_barrier()

        sl = pl.ds(base, ROWS_PER_TILE)
        pltpu.sync_copy(acc.at[sl], agg_out.at[c].at[sl])
        if with_cnt:
            @pl.when(c == 0)
            def _():
                pltpu.sync_copy(cntacc.at[sl], cnt_s_out.at[sl])

            @pl.when(c == 1)
            def _():
                pltpu.sync_copy(cntacc.at[sl], cnt_g_out.at[sl])

    return k


_sc_agg_cnt = _make_sc_agg(True)
_sc_agg = _make_sc_agg(False)


def _make_sc_pair_gather():
    scratch = [
        pltpu.VMEM((NCF, CHF), i32),
        pltpu.VMEM((RINGF, CHF, DP), i32),
        pltpu.SemaphoreType.DMA,
        pltpu.SemaphoreType.DMA,
    ]
    out_type = [jax.ShapeDtypeStruct((BPAD, DP), i32),
                jax.ShapeDtypeStruct((BPAD, DP), i32)]

    @functools.partial(pl.kernel, out_type=out_type, mesh=_MESH,
                       scratch_types=scratch)
    def k(zc, zd, r1h, r2h, zcg, zdg, rv, rb, sem_g, sem_w):
        c = lax.axis_index("c")
        s = lax.axis_index("s")
        pbase = s * PPTF

        @pl.when(c == 0)
        def _():
            pltpu.sync_copy(r1h.at[s], rv)

        @pl.when(c == 1)
        def _():
            pltpu.sync_copy(r2h.at[s], rv)

        def gdesc(kk, u, table):
            return pltpu.make_async_copy(table.at[rv.at[kk]], rb.at[u],
                                         sem_g)

        def wdesc(kk, u, outt):
            return pltpu.make_async_copy(
                rb.at[u], outt.at[pl.ds(pbase + kk * CHF, CHF)], sem_w)

        AHEADF = RINGF - 3

        def run(table, outt):
            for p in range(AHEADF):
                gdesc(p, p, table).start()

            def stage(kk, u):
                gdesc(kk, u, table).wait()
                wdesc(kk, u, outt).start()

                @pl.when(kk >= 3)
                def _():
                    wdesc(kk - 3, (u - 3) % RINGF, outt).wait()

                @pl.when(kk < NCF - AHEADF)
                def _():
                    gdesc(kk + AHEADF, (u + AHEADF) % RINGF, table).start()

            def step(j, carry):
                for u in range(RINGF):
                    stage(j * RINGF + u, u)
                return carry
            nfull = NCF // RINGF
            lax.fori_loop(0, nfull, step, 0)
            for kk in range(nfull * RINGF, NCF):
                stage(kk, kk % RINGF)
            wdesc(NCF - 3, (NCF - 3) % RINGF, outt).wait()
            wdesc(NCF - 2, (NCF - 2) % RINGF, outt).wait()
            wdesc(NCF - 1, (NCF - 1) % RINGF, outt).wait()

        @pl.when(c == 0)
        def _():
            run(zc, zcg)

        @pl.when(c == 1)
        def _():
            run(zd, zdg)

    return k


_sc_pair_gather = _make_sc_pair_gather()


def _tc_dot(a, b):
    BR = 2048
    grid = (BPAD // BR,)

    def body(a_ref, b_ref, o_ref):
        au = lax.bitcast_convert_type(a_ref[...], jnp.uint32)
        bu = lax.bitcast_convert_type(b_ref[...], jnp.uint32)
        hi_mask = jnp.uint32(0xFFFF0000)
        a_lo = lax.bitcast_convert_type(au << 16, f32)
        a_hi = lax.bitcast_convert_type(au & hi_mask, f32)
        b_lo = lax.bitcast_convert_type(bu << 16, f32)
        b_hi = lax.bitcast_convert_type(bu & hi_mask, f32)
        o_ref[...] = jnp.sum(a_lo * b_lo + a_hi * b_hi, axis=1,
                             keepdims=True)

    return pl.pallas_call(
        body,
        grid=grid,
        in_specs=[pl.BlockSpec((BR, DP), lambda i: (i, 0)),
                  pl.BlockSpec((BR, DP), lambda i: (i, 0))],
        out_specs=pl.BlockSpec((BR, 1), lambda i: (i, 0)),
        out_shape=jax.ShapeDtypeStruct((BPAD, 1), f32),
    )(a, b)


def _tc_lin(agg, cnt, x, Wl, Wr, bvec, relu, split_out):
    BR = 512
    grid = (NPAD // BR,)

    def body(agg_ref, cnt_ref, x_ref, wl_ref, wr_ref, b_ref, o_ref):
        scale = 1.0 / jnp.maximum(cnt_ref[...], 1.0)
        m_lo = (agg_ref[0] * scale).astype(bf16)
        m_hi = (agg_ref[1] * scale).astype(bf16)
        acc = jnp.dot(m_lo, wl_ref[:DH, :], preferred_element_type=f32)
        acc += jnp.dot(m_hi, wl_ref[DH:, :], preferred_element_type=f32)
        acc += jnp.dot(x_ref[0].astype(bf16), wr_ref[:DH, :],
                       preferred_element_type=f32)
        acc += jnp.dot(x_ref[1].astype(bf16), wr_ref[DH:, :],
                       preferred_element_type=f32)
        acc += b_ref[...]
        if relu:
            acc = jnp.maximum(acc, 0.0)
        if split_out:
            o_ref[0] = acc[:, :DH]
            o_ref[1] = acc[:, DH:]
        else:
            u = lax.bitcast_convert_type(acc, jnp.uint32)
            r = (u + 0x7FFF + ((u >> 16) & 1)) >> 16
            pk = r[:, :DH] | (r[:, DH:] << 16)
            o_ref[...] = lax.bitcast_convert_type(pk, i32)

    if split_out:
        out_shape = jax.ShapeDtypeStruct((2, NPAD, DH), f32)
        out_spec = pl.BlockSpec((2, BR, DH), lambda i: (0, i, 0))
    else:
        out_shape = jax.ShapeDtypeStruct((NPAD, DP), i32)
        out_spec = pl.BlockSpec((BR, DP), lambda i: (i, 0))
    return pl.pallas_call(
        body,
        grid=grid,
        in_specs=[pl.BlockSpec((2, BR, DH), lambda i: (0, i, 0)),
                  pl.BlockSpec((BR, 1), lambda i: (i, 0)),
                  pl.BlockSpec((2, BR, DH), lambda i: (0, i, 0)),
                  pl.BlockSpec((D, D), lambda i: (0, 0)),
                  pl.BlockSpec((D, D), lambda i: (0, 0)),
                  pl.BlockSpec((1, D), lambda i: (0, 0))],
        out_specs=out_spec,
        out_shape=out_shape,
    )(agg, cnt, x, Wl.astype(bf16), Wr.astype(bf16), bvec)


def kernel(x_cell, x_drug, edge_index, edge_label_index,
           W1_cd_l, b1_cd, W1_cd_r, W1_dc_l, b1_dc, W1_dc_r,
           W2_cd_l, b2_cd, W2_cd_r, W2_dc_l, b2_dc, W2_dc_r):
    src = edge_index[0].astype(i32)
    dst = edge_index[1].astype(i32)
    n_epad = EPAD - E
    junk = N + (jnp.arange(n_epad, dtype=i32) % (NPAD - N))
    src_p = jnp.concatenate([src, junk]).reshape(16, NCHUNK, CH)
    dst_p = jnp.concatenate([dst, junk]).reshape(16, NCHUNK, CH)

    xc = jnp.pad(x_cell, ((0, NPAD - N), (0, 0)))
    xd = jnp.pad(x_drug, ((0, NPAD - N), (0, 0)))
    xc_s = jnp.stack([xc[:, :DH], xc[:, DH:]])
    xd_s = jnp.stack([xd[:, :DH], xd[:, DH:]])

    agg1d, cnt_d, cnt_c = _sc_agg_cnt(xc_s, src_p, dst_p)
    (agg1c,) = _sc_agg(xd_s, dst_p, src_p)
    cnt_d2 = cnt_d.reshape(NPAD, 1)
    cnt_c2 = cnt_c.reshape(NPAD, 1)

    h_drug = _tc_lin(agg1d, cnt_d2, xd_s, W1_cd_l, W1_cd_r,
                     b1_cd.reshape(1, D), True, True)
    h_cell = _tc_lin(agg1c, cnt_c2, xc_s, W1_dc_l, W1_dc_r,
                     b1_dc.reshape(1, D), True, True)

    (agg2d,) = _sc_agg(h_cell, src_p, dst_p)
    (agg2c,) = _sc_agg(h_drug, dst_p, src_p)

    z_drug = _tc_lin(agg2d, cnt_d2, h_drug, W2_cd_l, W2_cd_r,
                     b2_cd.reshape(1, D), False, False)
    z_cell = _tc_lin(agg2c, cnt_c2, h_cell, W2_dc_l, W2_dc_r,
                     b2_dc.reshape(1, D), False, False)

    r1 = edge_label_index[0].astype(i32)
    r2 = edge_label_index[1].astype(i32)
    n_bpad = BPAD - B
    padp = jnp.arange(n_bpad, dtype=i32) % N
    r1p = jnp.concatenate([r1, padp]).reshape(16, NCF, CHF)
    r2p = jnp.concatenate([r2, padp]).reshape(16, NCF, CHF)

    zcg_i, zdg_i = _sc_pair_gather(z_cell, z_drug, r1p, r2p)
    out = _tc_dot(zcg_i, zdg_i)
    return out.reshape(BPAD)[:B]


# speedup vs baseline: 1.0066x; 1.0066x over previous
import functools

import jax
import jax.numpy as jnp
from jax import lax
from jax.experimental import pallas as pl
from jax.experimental.pallas import tpu as pltpu
from jax.experimental.pallas import tpu_sc as plsc

N = 10000
NPAD = 10240
D = 256
DH = 128
E = 160000
EPAD = 163840
CH = 32
NCHUNK = 320
NLOAD = 40
RING = 8
RINGF = 7
ROWS_PER_TILE = NPAD // 16
B = 50000
BPAD = 53248
PPTF = 3328
CHF = 64
NCF = 52
DP = 128

f32 = jnp.float32
bf16 = jnp.bfloat16
i32 = jnp.int32

_MESH = plsc.VectorSubcoreMesh(core_axis_name="c", subcore_axis_name="s")


def _make_sc_agg(with_cnt):
    out_type = [jax.ShapeDtypeStruct((2, NPAD, DH), f32)]
    if with_cnt:
        out_type += [jax.ShapeDtypeStruct((NPAD,), f32),
                     jax.ShapeDtypeStruct((NPAD,), f32)]
    scratch = [
        pltpu.VMEM_SHARED((NPAD, DH), f32),
        pltpu.VMEM((NLOAD, CH), i32),
        pltpu.VMEM((NLOAD, CH), i32),
        pltpu.VMEM((RING, CH, DH), f32),
        pltpu.SemaphoreType.DMA,
        pltpu.SemaphoreType.DMA,
    ]
    if with_cnt:
        scratch += [
            pltpu.VMEM_SHARED((NPAD,), f32),
            pltpu.VMEM((CH,), f32),
            pltpu.SemaphoreType.DMA,
        ]

    @functools.partial(pl.kernel, out_type=out_type, mesh=_MESH,
                       scratch_types=scratch)
    def k(table, gidx, sidx, *rest):
        if with_cnt:
            agg_out, cnt_s_out, cnt_g_out = rest[0], rest[1], rest[2]
            acc, gv, sv, rb, sem_g, sem_s, cntacc, ones_v, sem_c = rest[3:]
        else:
            agg_out = rest[0]
            acc, gv, sv, rb, sem_g, sem_s = rest[1:]
            cntacc = ones_v = sem_c = None
        c = lax.axis_index("c")
        s = lax.axis_index("s")
        base = s * ROWS_PER_TILE

        def zrow(j, carry):
            rb[0, j // 8, pl.ds((j % 8) * 16, 16)] = jnp.zeros((16,), f32)
            return carry
        lax.fori_loop(0, (CH * DH) // 16, zrow, 0)
        nz = ROWS_PER_TILE // CH
        for q in range(nz):
            pltpu.make_async_copy(
                rb.at[0], acc.at[pl.ds(base + q * CH, CH)], sem_g).start()
        for q in range(nz):
            pltpu.make_async_copy(
                rb.at[0], acc.at[pl.ds(base + q * CH, CH)], sem_g).wait()
        if with_cnt:
            for q in range(ROWS_PER_TILE // DH):
                pltpu.sync_copy(rb.at[0].at[0],
                                cntacc.at[pl.ds(base + q * DH, DH)])
            for q in range(CH // 16):
                ones_v[pl.ds(q * 16, 16)] = jnp.ones((16,), f32)

        def gdesc(kk, u):
            return pltpu.make_async_copy(table.at[c].at[gv.at[kk]],
                                         rb.at[u], sem_g)

        def sdesc(kk, u):
            return pltpu.make_async_copy(rb.at[u], acc.at[sv.at[kk]], sem_s)

        def cdesc(kk, use_sv):
            idxv = sv if use_sv else gv
            return pltpu.make_async_copy(ones_v, cntacc.at[idxv.at[kk]],
                                         sem_c)

        AHEAD = RING - 3

        def step(j, carry):
            for u in range(RING):
                kk = j * RING + u
                gdesc(kk, u).wait()
                sdesc(kk, u).start(add=True)
                if with_cnt:
                    @pl.when(c == 0)
                    def _():
                        cdesc(kk, True).start(add=True)

                    @pl.when(c == 1)
                    def _():
                        cdesc(kk, False).start(add=True)

                    @pl.when(kk >= 1)
                    def _():
                        @pl.when(c == 0)
                        def _():
                            cdesc(kk - 1, True).wait()

                        @pl.when(c == 1)
                        def _():
                            cdesc(kk - 1, False).wait()

                @pl.when(kk >= 3)
                def _():
                    sdesc(kk - 3, (u - 3) % RING).wait()

                @pl.when(kk < NLOAD - AHEAD)
                def _():
                    gdesc(kk + AHEAD, (u + AHEAD) % RING).start()
            return carry

        for h in range(NCHUNK // NLOAD):
            pltpu.sync_copy(gidx.at[s].at[pl.ds(h * NLOAD, NLOAD)], gv)
            pltpu.sync_copy(sidx.at[s].at[pl.ds(h * NLOAD, NLOAD)], sv)
            if h == 0:
                plsc.subcore_barrier()
            for p in range(AHEAD):
                gdesc(p, p).start()
            lax.fori_loop(0, NLOAD // RING, step, 0)
            sdesc(NLOAD - 3, (NLOAD - 3) % RING).wait()
            sdesc(NLOAD - 2, (NLOAD - 2) % RING).wait()
            sdesc(NLOAD - 1, (NLOAD - 1) % RING).wait()
            if with_cnt:
                @pl.when(c == 0)
                def _():
                    cdesc(NLOAD - 1, True).wait()

                @pl.when(c == 1)
                def _():
                    cdesc(NLOAD - 1, False).wait()
        plsc.subcore_barrier()

        sl = pl.ds(base, ROWS_PER_TILE)
        pltpu.sync_copy(acc.at[sl], agg_out.at[c].at[sl])
        if with_cnt:
            @pl.when(c == 0)
            def _():
                pltpu.sync_copy(cntacc.at[sl], cnt_s_out.at[sl])

            @pl.when(c == 1)
            def _():
                pltpu.sync_copy(cntacc.at[sl], cnt_g_out.at[sl])

    return k


_sc_agg_cnt = _make_sc_agg(True)
_sc_agg = _make_sc_agg(False)


def _make_sc_pair_gather():
    scratch = [
        pltpu.VMEM((NCF, CHF), i32),
        pltpu.VMEM((RINGF, CHF, DP), i32),
        pltpu.SemaphoreType.DMA,
        pltpu.SemaphoreType.DMA,
    ]
    out_type = [jax.ShapeDtypeStruct((BPAD, DP), i32),
                jax.ShapeDtypeStruct((BPAD, DP), i32)]

    @functools.partial(pl.kernel, out_type=out_type, mesh=_MESH,
                       scratch_types=scratch)
    def k(zc, zd, r1h, r2h, zcg, zdg, rv, rb, sem_g, sem_w):
        c = lax.axis_index("c")
        s = lax.axis_index("s")
        pbase = s * PPTF

        @pl.when(c == 0)
        def _():
            pltpu.sync_copy(r1h.at[s], rv)

        @pl.when(c == 1)
        def _():
            pltpu.sync_copy(r2h.at[s], rv)

        def gdesc(kk, u, table):
            return pltpu.make_async_copy(table.at[rv.at[kk]], rb.at[u],
                                         sem_g)

        def wdesc(kk, u, outt):
            return pltpu.make_async_copy(
                rb.at[u], outt.at[pl.ds(pbase + kk * CHF, CHF)], sem_w)

        AHEADF = RINGF - 3

        def run(table, outt):
            for p in range(AHEADF):
                gdesc(p, p, table).start()

            def stage(kk, u):
                gdesc(kk, u, table).wait()
                wdesc(kk, u, outt).start()

                @pl.when(kk >= 3)
                def _():
                    wdesc(kk - 3, (u - 3) % RINGF, outt).wait()

                @pl.when(kk < NCF - AHEADF)
                def _():
                    gdesc(kk + AHEADF, (u + AHEADF) % RINGF, table).start()

            def step(j, carry):
                for u in range(RINGF):
                    stage(j * RINGF + u, u)
                return carry
            nfull = NCF // RINGF
            lax.fori_loop(0, nfull, step, 0)
            for kk in range(nfull * RINGF, NCF):
                stage(kk, kk % RINGF)
            wdesc(NCF - 3, (NCF - 3) % RINGF, outt).wait()
            wdesc(NCF - 2, (NCF - 2) % RINGF, outt).wait()
            wdesc(NCF - 1, (NCF - 1) % RINGF, outt).wait()

        @pl.when(c == 0)
        def _():
            run(zc, zcg)

        @pl.when(c == 1)
        def _():
            run(zd, zdg)

    return k


_sc_pair_gather = _make_sc_pair_gather()


def _tc_dot(a, b):
    BR = 2048
    grid = (BPAD // BR,)

    def body(a_ref, b_ref, o_ref):
        au = lax.bitcast_convert_type(a_ref[...], jnp.uint32)
        bu = lax.bitcast_convert_type(b_ref[...], jnp.uint32)
        hi_mask = jnp.uint32(0xFFFF0000)
        a_lo = lax.bitcast_convert_type(au << 16, f32)
        a_hi = lax.bitcast_convert_type(au & hi_mask, f32)
        b_lo = lax.bitcast_convert_type(bu << 16, f32)
        b_hi = lax.bitcast_convert_type(bu & hi_mask, f32)
        o_ref[...] = jnp.sum(a_lo * b_lo + a_hi * b_hi, axis=1,
                             keepdims=True)

    return pl.pallas_call(
        body,
        grid=grid,
        in_specs=[pl.BlockSpec((BR, DP), lambda i: (i, 0)),
                  pl.BlockSpec((BR, DP), lambda i: (i, 0))],
        out_specs=pl.BlockSpec((BR, 1), lambda i: (i, 0)),
        out_shape=jax.ShapeDtypeStruct((BPAD, 1), f32),
    )(a, b)


def _tc_lin(agg, cnt, x, Wl, Wr, bvec, relu, split_out):
    BR = 512
    grid = (NPAD // BR,)

    def body(agg_ref, cnt_ref, x_ref, wl_ref, wr_ref, b_ref, o_ref):
        scale = 1.0 / jnp.maximum(cnt_ref[...], 1.0)
        acc = jnp.dot(agg_ref[0] * scale, wl_ref[:DH, :],
                      preferred_element_type=f32)
        acc += jnp.dot(agg_ref[1] * scale, wl_ref[DH:, :],
                       preferred_element_type=f32)
        acc += jnp.dot(x_ref[0], wr_ref[:DH, :], preferred_element_type=f32)
        acc += jnp.dot(x_ref[1], wr_ref[DH:, :], preferred_element_type=f32)
        acc += b_ref[...]
        if relu:
            acc = jnp.maximum(acc, 0.0)
        if split_out:
            o_ref[0] = acc[:, :DH]
            o_ref[1] = acc[:, DH:]
        else:
            u = lax.bitcast_convert_type(acc, jnp.uint32)
            r = (u + 0x7FFF + ((u >> 16) & 1)) >> 16
            pk = r[:, :DH] | (r[:, DH:] << 16)
            o_ref[...] = lax.bitcast_convert_type(pk, i32)

    if split_out:
        out_shape = jax.ShapeDtypeStruct((2, NPAD, DH), f32)
        out_spec = pl.BlockSpec((2, BR, DH), lambda i: (0, i, 0))
    else:
        out_shape = jax.ShapeDtypeStruct((NPAD, DP), i32)
        out_spec = pl.BlockSpec((BR, DP), lambda i: (i, 0))
    return pl.pallas_call(
        body,
        grid=grid,
        in_specs=[pl.BlockSpec((2, BR, DH), lambda i: (0, i, 0)),
                  pl.BlockSpec((BR, 1), lambda i: (i, 0)),
                  pl.BlockSpec((2, BR, DH), lambda i: (0, i, 0)),
                  pl.BlockSpec((D, D), lambda i: (0, 0)),
                  pl.BlockSpec((D, D), lambda i: (0, 0)),
                  pl.BlockSpec((1, D), lambda i: (0, 0))],
        out_specs=out_spec,
        out_shape=out_shape,
    )(agg, cnt, x, Wl, Wr, bvec)


def kernel(x_cell, x_drug, edge_index, edge_label_index,
           W1_cd_l, b1_cd, W1_cd_r, W1_dc_l, b1_dc, W1_dc_r,
           W2_cd_l, b2_cd, W2_cd_r, W2_dc_l, b2_dc, W2_dc_r):
    src = edge_index[0].astype(i32)
    dst = edge_index[1].astype(i32)
    n_epad = EPAD - E
    junk = N + (jnp.arange(n_epad, dtype=i32) % (NPAD - N))
    src_p = jnp.concatenate([src, junk]).reshape(16, NCHUNK, CH)
    dst_p = jnp.concatenate([dst, junk]).reshape(16, NCHUNK, CH)

    xc = jnp.pad(x_cell, ((0, NPAD - N), (0, 0)))
    xd = jnp.pad(x_drug, ((0, NPAD - N), (0, 0)))
    xc_s = jnp.stack([xc[:, :DH], xc[:, DH:]])
    xd_s = jnp.stack([xd[:, :DH], xd[:, DH:]])

    agg1d, cnt_d, cnt_c = _sc_agg_cnt(xc_s, src_p, dst_p)
    (agg1c,) = _sc_agg(xd_s, dst_p, src_p)
    cnt_d2 = cnt_d.reshape(NPAD, 1)
    cnt_c2 = cnt_c.reshape(NPAD, 1)

    h_drug = _tc_lin(agg1d, cnt_d2, xd_s, W1_cd_l, W1_cd_r,
                     b1_cd.reshape(1, D), True, True)
    h_cell = _tc_lin(agg1c, cnt_c2, xc_s, W1_dc_l, W1_dc_r,
                     b1_dc.reshape(1, D), True, True)

    (agg2c,) = _sc_agg(h_drug, dst_p, src_p)
    (agg2d,) = _sc_agg(h_cell, src_p, dst_p)

    z_cell = _tc_lin(agg2c, cnt_c2, h_cell, W2_dc_l, W2_dc_r,
                     b2_dc.reshape(1, D), False, False)
    z_drug = _tc_lin(agg2d, cnt_d2, h_drug, W2_cd_l, W2_cd_r,
                     b2_cd.reshape(1, D), False, False)

    r1 = edge_label_index[0].astype(i32)
    r2 = edge_label_index[1].astype(i32)
    n_bpad = BPAD - B
    padp = jnp.arange(n_bpad, dtype=i32) % N
    r1p = jnp.concatenate([r1, padp]).reshape(16, NCF, CHF)
    r2p = jnp.concatenate([r2, padp]).reshape(16, NCF, CHF)

    zcg_i, zdg_i = _sc_pair_gather(z_cell, z_drug, r1p, r2p)
    out = _tc_dot(zcg_i, zdg_i)
    return out.reshape(BPAD)[:B]


# speedup vs baseline: 1.0180x; 1.0113x over previous
import functools

import jax
import jax.numpy as jnp
from jax import lax
from jax.experimental import pallas as pl
from jax.experimental.pallas import tpu as pltpu
from jax.experimental.pallas import tpu_sc as plsc

N = 10000
NPAD = 10240
D = 256
DH = 128
E = 160000
EPAD = 163840
CH = 32
NCHUNK = 320
NLOAD = 40
RING = 8
RINGF = 7
ROWS_PER_TILE = NPAD // 16
B = 50000
BPAD = 53248
PPTF = 3328
CHF = 64
NCF = 52
DP = 128

f32 = jnp.float32
bf16 = jnp.bfloat16
i32 = jnp.int32

_MESH = plsc.VectorSubcoreMesh(core_axis_name="c", subcore_axis_name="s")


def _make_sc_agg(with_cnt):
    out_type = [jax.ShapeDtypeStruct((2, NPAD, DH), f32)]
    if with_cnt:
        out_type += [jax.ShapeDtypeStruct((NPAD,), f32),
                     jax.ShapeDtypeStruct((NPAD,), f32)]
    scratch = [
        pltpu.VMEM_SHARED((NPAD, DH), f32),
        pltpu.VMEM((NLOAD, CH), i32),
        pltpu.VMEM((NLOAD, CH), i32),
        pltpu.VMEM((RING, CH, DH), f32),
        pltpu.SemaphoreType.DMA,
        pltpu.SemaphoreType.DMA,
    ]
    if with_cnt:
        scratch += [
            pltpu.VMEM_SHARED((NPAD,), f32),
            pltpu.VMEM((CH,), f32),
            pltpu.SemaphoreType.DMA,
        ]

    @functools.partial(pl.kernel, out_type=out_type, mesh=_MESH,
                       scratch_types=scratch)
    def k(table, gidx, sidx, *rest):
        if with_cnt:
            agg_out, cnt_s_out, cnt_g_out = rest[0], rest[1], rest[2]
            acc, gv, sv, rb, sem_g, sem_s, cntacc, ones_v, sem_c = rest[3:]
        else:
            agg_out = rest[0]
            acc, gv, sv, rb, sem_g, sem_s = rest[1:]
            cntacc = ones_v = sem_c = None
        c = lax.axis_index("c")
        s = lax.axis_index("s")
        base = s * ROWS_PER_TILE

        def zrow(j, carry):
            rb[0, j // 8, pl.ds((j % 8) * 16, 16)] = jnp.zeros((16,), f32)
            return carry
        lax.fori_loop(0, (CH * DH) // 16, zrow, 0)
        nz = ROWS_PER_TILE // CH
        for q in range(nz):
            pltpu.make_async_copy(
                rb.at[0], acc.at[pl.ds(base + q * CH, CH)], sem_g).start()
        for q in range(nz):
            pltpu.make_async_copy(
                rb.at[0], acc.at[pl.ds(base + q * CH, CH)], sem_g).wait()
        if with_cnt:
            for q in range(ROWS_PER_TILE // DH):
                pltpu.sync_copy(rb.at[0].at[0],
                                cntacc.at[pl.ds(base + q * DH, DH)])
            for q in range(CH // 16):
                ones_v[pl.ds(q * 16, 16)] = jnp.ones((16,), f32)

        def gdesc(kk, u):
            return pltpu.make_async_copy(table.at[c].at[gv.at[kk]],
                                         rb.at[u], sem_g)

        def sdesc(kk, u):
            return pltpu.make_async_copy(rb.at[u], acc.at[sv.at[kk]], sem_s)

        def cdesc(kk, use_sv):
            idxv = sv if use_sv else gv
            return pltpu.make_async_copy(ones_v, cntacc.at[idxv.at[kk]],
                                         sem_c)

        AHEAD = RING - 3

        def step(j, carry):
            for u in range(RING):
                kk = j * RING + u
                gdesc(kk, u).wait()
                sdesc(kk, u).start(add=True)
                if with_cnt:
                    @pl.when(c == 0)
                    def _():
                        cdesc(kk, True).start(add=True)

                    @pl.when(c == 1)
                    def _():
                        cdesc(kk, False).start(add=True)

                    @pl.when(kk >= 1)
                    def _():
                        @pl.when(c == 0)
                        def _():
                            cdesc(kk - 1, True).wait()

                        @pl.when(c == 1)
                        def _():
                            cdesc(kk - 1, False).wait()

                @pl.when(kk >= 3)
                def _():
                    sdesc(kk - 3, (u - 3) % RING).wait()

                @pl.when(kk < NLOAD - AHEAD)
                def _():
                    gdesc(kk + AHEAD, (u + AHEAD) % RING).start()
            return carry

        for h in range(NCHUNK // NLOAD):
            pltpu.sync_copy(gidx.at[s].at[pl.ds(h * NLOAD, NLOAD)], gv)
            pltpu.sync_copy(sidx.at[s].at[pl.ds(h * NLOAD, NLOAD)], sv)
            if h == 0:
                plsc.subcore_barrier()
            for p in range(AHEAD):
                gdesc(p, p).start()
            lax.fori_loop(0, NLOAD // RING, step, 0)
            sdesc(NLOAD - 3, (NLOAD - 3) % RING).wait()
            sdesc(NLOAD - 2, (NLOAD - 2) % RING).wait()
            sdesc(NLOAD - 1, (NLOAD - 1) % RING).wait()
            if with_cnt:
                @pl.when(c == 0)
                def _():
                    cdesc(NLOAD - 1, True).wait()

                @pl.when(c == 1)
                def _():
                    cdesc(NLOAD - 1, False).wait()
        plsc.subcore_barrier()

        sl = pl.ds(base, ROWS_PER_TILE)
        pltpu.sync_copy(acc.at[sl], agg_out.at[c].at[sl])
        if with_cnt:
            @pl.when(c == 0)
            def _():
                pltpu.sync_copy(cntacc.at[sl], cnt_s_out.at[sl])

            @pl.when(c == 1)
            def _():
                pltpu.sync_copy(cntacc.at[sl], cnt_g_out.at[sl])

    return k


_sc_agg_cnt = _make_sc_agg(True)
_sc_agg = _make_sc_agg(False)


def _make_sc_pair_gather(ncf):
    scratch = [
        pltpu.VMEM((ncf, CHF), i32),
        pltpu.VMEM((RINGF, CHF, DP), i32),
        pltpu.SemaphoreType.DMA,
        pltpu.SemaphoreType.DMA,
    ]
    out_type = [jax.ShapeDtypeStruct((16 * ncf * CHF, DP), i32),
                jax.ShapeDtypeStruct((16 * ncf * CHF, DP), i32)]

    @functools.partial(pl.kernel, out_type=out_type, mesh=_MESH,
                       scratch_types=scratch)
    def k(zc, zd, r1h, r2h, zcg, zdg, rv, rb, sem_g, sem_w):
        c = lax.axis_index("c")
        s = lax.axis_index("s")
        pbase = s * (ncf * CHF)

        @pl.when(c == 0)
        def _():
            pltpu.sync_copy(r1h.at[s], rv)

        @pl.when(c == 1)
        def _():
            pltpu.sync_copy(r2h.at[s], rv)

        def gdesc(kk, u, table):
            return pltpu.make_async_copy(table.at[rv.at[kk]], rb.at[u],
                                         sem_g)

        def wdesc(kk, u, outt):
            return pltpu.make_async_copy(
                rb.at[u], outt.at[pl.ds(pbase + kk * CHF, CHF)], sem_w)

        AHEADF = RINGF - 3

        def run(table, outt):
            for p in range(AHEADF):
                gdesc(p, p, table).start()

            def stage(kk, u):
                gdesc(kk, u, table).wait()
                wdesc(kk, u, outt).start()

                @pl.when(kk >= 3)
                def _():
                    wdesc(kk - 3, (u - 3) % RINGF, outt).wait()

                @pl.when(kk < ncf - AHEADF)
                def _():
                    gdesc(kk + AHEADF, (u + AHEADF) % RINGF, table).start()

            def step(j, carry):
                for u in range(RINGF):
                    stage(j * RINGF + u, u)
                return carry
            nfull = ncf // RINGF
            lax.fori_loop(0, nfull, step, 0)
            for kk in range(nfull * RINGF, ncf):
                stage(kk, kk % RINGF)
            wdesc(ncf - 3, (ncf - 3) % RINGF, outt).wait()
            wdesc(ncf - 2, (ncf - 2) % RINGF, outt).wait()
            wdesc(ncf - 1, (ncf - 1) % RINGF, outt).wait()

        @pl.when(c == 0)
        def _():
            run(zc, zcg)

        @pl.when(c == 1)
        def _():
            run(zd, zdg)

    return k


_sc_pair_gather_h = _make_sc_pair_gather(NCF // 2)


def _tc_dot(a, b):
    BR = 2048
    nrows = a.shape[0]
    grid = (nrows // BR,)

    def body(a_ref, b_ref, o_ref):
        au = lax.bitcast_convert_type(a_ref[...], jnp.uint32)
        bu = lax.bitcast_convert_type(b_ref[...], jnp.uint32)
        hi_mask = jnp.uint32(0xFFFF0000)
        a_lo = lax.bitcast_convert_type(au << 16, f32)
        a_hi = lax.bitcast_convert_type(au & hi_mask, f32)
        b_lo = lax.bitcast_convert_type(bu << 16, f32)
        b_hi = lax.bitcast_convert_type(bu & hi_mask, f32)
        o_ref[...] = jnp.sum(a_lo * b_lo + a_hi * b_hi, axis=1,
                             keepdims=True)

    return pl.pallas_call(
        body,
        grid=grid,
        in_specs=[pl.BlockSpec((BR, DP), lambda i: (i, 0)),
                  pl.BlockSpec((BR, DP), lambda i: (i, 0))],
        out_specs=pl.BlockSpec((BR, 1), lambda i: (i, 0)),
        out_shape=jax.ShapeDtypeStruct((nrows, 1), f32),
    )(a, b)


def _tc_lin(agg, cnt, x, Wl, Wr, bvec, relu, split_out):
    BR = 1024
    grid = (NPAD // BR,)

    def body(agg_ref, cnt_ref, x_ref, wl_ref, wr_ref, b_ref, o_ref):
        scale = 1.0 / jnp.maximum(cnt_ref[...], 1.0)
        acc = jnp.dot(agg_ref[0] * scale, wl_ref[:DH, :],
                      preferred_element_type=f32)
        acc += jnp.dot(agg_ref[1] * scale, wl_ref[DH:, :],
                       preferred_element_type=f32)
        acc += jnp.dot(x_ref[0], wr_ref[:DH, :], preferred_element_type=f32)
        acc += jnp.dot(x_ref[1], wr_ref[DH:, :], preferred_element_type=f32)
        acc += b_ref[...]
        if relu:
            acc = jnp.maximum(acc, 0.0)
        if split_out:
            o_ref[0] = acc[:, :DH]
            o_ref[1] = acc[:, DH:]
        else:
            u = lax.bitcast_convert_type(acc, jnp.uint32)
            r = (u + 0x7FFF + ((u >> 16) & 1)) >> 16
            pk = r[:, :DH] | (r[:, DH:] << 16)
            o_ref[...] = lax.bitcast_convert_type(pk, i32)

    if split_out:
        out_shape = jax.ShapeDtypeStruct((2, NPAD, DH), f32)
        out_spec = pl.BlockSpec((2, BR, DH), lambda i: (0, i, 0))
    else:
        out_shape = jax.ShapeDtypeStruct((NPAD, DP), i32)
        out_spec = pl.BlockSpec((BR, DP), lambda i: (i, 0))
    return pl.pallas_call(
        body,
        grid=grid,
        in_specs=[pl.BlockSpec((2, BR, DH), lambda i: (0, i, 0)),
                  pl.BlockSpec((BR, 1), lambda i: (i, 0)),
                  pl.BlockSpec((2, BR, DH), lambda i: (0, i, 0)),
                  pl.BlockSpec((D, D), lambda i: (0, 0)),
                  pl.BlockSpec((D, D), lambda i: (0, 0)),
                  pl.BlockSpec((1, D), lambda i: (0, 0))],
        out_specs=out_spec,
        out_shape=out_shape,
    )(agg, cnt, x, Wl, Wr, bvec)


def kernel(x_cell, x_drug, edge_index, edge_label_index,
           W1_cd_l, b1_cd, W1_cd_r, W1_dc_l, b1_dc, W1_dc_r,
           W2_cd_l, b2_cd, W2_cd_r, W2_dc_l, b2_dc, W2_dc_r):
    src = edge_index[0].astype(i32)
    dst = edge_index[1].astype(i32)
    n_epad = EPAD - E
    junk = N + (jnp.arange(n_epad, dtype=i32) % (NPAD - N))
    src_p = jnp.concatenate([src, junk]).reshape(16, NCHUNK, CH)
    dst_p = jnp.concatenate([dst, junk]).reshape(16, NCHUNK, CH)

    xc = jnp.pad(x_cell, ((0, NPAD - N), (0, 0)))
    xd = jnp.pad(x_drug, ((0, NPAD - N), (0, 0)))
    xc_s = jnp.stack([xc[:, :DH], xc[:, DH:]])
    xd_s = jnp.stack([xd[:, :DH], xd[:, DH:]])

    agg1d, cnt_d, cnt_c = _sc_agg_cnt(xc_s, src_p, dst_p)
    (agg1c,) = _sc_agg(xd_s, dst_p, src_p)
    cnt_d2 = cnt_d.reshape(NPAD, 1)
    cnt_c2 = cnt_c.reshape(NPAD, 1)

    h_drug = _tc_lin(agg1d, cnt_d2, xd_s, W1_cd_l, W1_cd_r,
                     b1_cd.reshape(1, D), True, True)
    h_cell = _tc_lin(agg1c, cnt_c2, xc_s, W1_dc_l, W1_dc_r,
                     b1_dc.reshape(1, D), True, True)

    (agg2c,) = _sc_agg(h_drug, dst_p, src_p)
    (agg2d,) = _sc_agg(h_cell, src_p, dst_p)

    z_cell = _tc_lin(agg2c, cnt_c2, h_cell, W2_dc_l, W2_dc_r,
                     b2_dc.reshape(1, D), False, False)
    z_drug = _tc_lin(agg2d, cnt_d2, h_drug, W2_cd_l, W2_cd_r,
                     b2_cd.reshape(1, D), False, False)

    r1 = edge_label_index[0].astype(i32)
    r2 = edge_label_index[1].astype(i32)
    n_bpad = BPAD - B
    padp = jnp.arange(n_bpad, dtype=i32) % N
    r1p = jnp.concatenate([r1, padp]).reshape(16, NCF, CHF)
    r2p = jnp.concatenate([r2, padp]).reshape(16, NCF, CHF)
    nh = NCF // 2
    za_i, zb_i = _sc_pair_gather_h(z_cell, z_drug,
                                   r1p[:, :nh], r2p[:, :nh])
    zc_i, zd_i = _sc_pair_gather_h(z_cell, z_drug,
                                   r1p[:, nh:], r2p[:, nh:])
    out_a = _tc_dot(za_i, zb_i)
    out_b = _tc_dot(zc_i, zd_i)
    out = jnp.concatenate([out_a.reshape(16, nh * CHF),
                           out_b.reshape(16, nh * CHF)], axis=1)
    return out.reshape(BPAD)[:B]
